# baseline (speedup 1.0000x reference)
import jax
import jax.numpy as jnp
from jax import lax
from jax.experimental import pallas as pl
from jax.experimental.pallas import tpu as pltpu
from jax.experimental.pallas import tpu_sc as plsc

N = 10000
E = 160000
D = 200
DG = 256
DH = 128
NREL = 3
CHUNK = 128
GATE_CHUNKS = 40
EP = 32 * GATE_CHUNKS * CHUNK
SCAT_CHUNKS = EP // (16 * CHUNK)
AGG_ROWS = 11000
STRIPE = 688
STRIPE_LAST = AGG_ROWS - 15 * STRIPE
DUMP_ROW = 10008

_BD = 1000


def _pad_cols(v, width):
    return jnp.concatenate(
        [v, jnp.zeros((v.shape[0], width - v.shape[1]), jnp.float32)], axis=1)


def _dense_products(h, w_ref, wc_ref, aw_ref, lw_ref, emb_ref, ab_ref,
                    xra_ref, xrb_ref, ps_ref, pd_ref, lp_ref, cd_ref):
    hw = [jnp.dot(h, w_ref[b], preferred_element_type=jnp.float32)
          for b in range(4)]
    for r in range(NREL):
        xr_r = (wc_ref[r, 0] * hw[0] + wc_ref[r, 1] * hw[1]
                + wc_ref[r, 2] * hw[2] + wc_ref[r, 3] * hw[3])
        xra_ref[r, :, :] = xr_r[:, 0:DH]
        xrb_ref[r, :, :] = _pad_cols(xr_r[:, DH:D], DH)
    ps_ref[...] = _pad_cols(jnp.dot(h, aw_ref[0:D, :],
                                    preferred_element_type=jnp.float32), DG)
    pd_ref[...] = _pad_cols(jnp.dot(h, aw_ref[D:2 * D, :],
                                    preferred_element_type=jnp.float32), DG)
    lp_ref[...] = jnp.dot(h, lw_ref[...], preferred_element_type=jnp.float32)
    emb = emb_ref[0:NREL, :]
    ca = jnp.dot(emb, aw_ref[2 * D:2 * D + 32, :],
                 preferred_element_type=jnp.float32)
    cb = jnp.dot(emb, aw_ref[2 * D + 32:2 * D + 64, :],
                 preferred_element_type=jnp.float32)
    cd = ca[:, None, :] + cb[None, :, :] + ab_ref[0, :][None, None, :]
    cd16 = jnp.concatenate(
        [cd.reshape(9, D), jnp.zeros((16 - 9, D), jnp.float32)], axis=0)
    cd_ref[...] = _pad_cols(cd16, DG)


def _dense1_body(h_ref, w_ref, wc_ref, aw_ref, lw_ref, emb_ref, ab_ref,
                 xra_ref, xrb_ref, ps_ref, pd_ref, lp_ref, cd_ref):
    _dense_products(h_ref[...], w_ref, wc_ref, aw_ref, lw_ref, emb_ref,
                    ab_ref, xra_ref, xrb_ref, ps_ref, pd_ref, lp_ref, cd_ref)


def _prev_h(pa_ref, pb_ref, lprev_ref):
    return jnp.maximum(
        jnp.concatenate([pa_ref[...], pb_ref[:, 0:D - DH]], axis=1)
        + lprev_ref[...], 0.0)


def _dense2_body(pa_ref, pb_ref, lprev_ref, w_ref, wc_ref, aw_ref, lw_ref,
                 emb_ref, ab_ref, xra_ref, xrb_ref, ps_ref, pd_ref, lp_ref,
                 cd_ref):
    _dense_products(_prev_h(pa_ref, pb_ref, lprev_ref), w_ref, wc_ref, aw_ref,
                    lw_ref, emb_ref, ab_ref, xra_ref, xrb_ref, ps_ref, pd_ref,
                    lp_ref, cd_ref)


def _final_body(pa_ref, pb_ref, lprev_ref, out_ref):
    out_ref[...] = _prev_h(pa_ref, pb_ref, lprev_ref)


def _dense_out_shapes():
    return [
        jax.ShapeDtypeStruct((NREL, N, DH), jnp.float32),
        jax.ShapeDtypeStruct((NREL, N, DH), jnp.float32),
        jax.ShapeDtypeStruct((N, DG), jnp.float32),
        jax.ShapeDtypeStruct((N, DG), jnp.float32),
        jax.ShapeDtypeStruct((N, D), jnp.float32),
        jax.ShapeDtypeStruct((16, DG), jnp.float32),
    ]


def _dense_out_specs():
    return [
        pl.BlockSpec((NREL, _BD, DH), lambda i: (0, i, 0)),
        pl.BlockSpec((NREL, _BD, DH), lambda i: (0, i, 0)),
        pl.BlockSpec((_BD, DG), lambda i: (i, 0)),
        pl.BlockSpec((_BD, DG), lambda i: (i, 0)),
        pl.BlockSpec((_BD, D), lambda i: (i, 0)),
        pl.BlockSpec((16, DG), lambda i: (0, 0)),
    ]


def _weight_specs():
    return [
        pl.BlockSpec((4, D, D), lambda i: (0, 0, 0)),
        pl.BlockSpec(memory_space=pltpu.SMEM),
        pl.BlockSpec((2 * D + 64, D), lambda i: (0, 0)),
        pl.BlockSpec((D, D), lambda i: (0, 0)),
        pl.BlockSpec((8, 32), lambda i: (0, 0)),
        pl.BlockSpec((8, D), lambda i: (0, 0)),
    ]


def _parts_specs():
    return [
        pl.BlockSpec((_BD, DH), lambda i: (i, 0)),
        pl.BlockSpec((_BD, DH), lambda i: (AGG_ROWS // _BD + i, 0)),
    ]


def _dense1(h, weight, wcomp, aw, loop_w, emb8, ab8):
    return pl.pallas_call(
        _dense1_body,
        grid=(N // _BD,),
        in_specs=[pl.BlockSpec((_BD, D), lambda i: (i, 0))] + _weight_specs(),
        out_specs=_dense_out_specs(),
        out_shape=_dense_out_shapes(),
    )(h, weight, wcomp, aw, loop_w, emb8, ab8)


def _dense2(parts, lprev, weight, wcomp, aw, loop_w, emb8, ab8):
    return pl.pallas_call(
        _dense2_body,
        grid=(N // _BD,),
        in_specs=_parts_specs() + [
            pl.BlockSpec((_BD, D), lambda i: (i, 0)),
        ] + _weight_specs(),
        out_specs=_dense_out_specs(),
        out_shape=_dense_out_shapes(),
    )(parts, parts, lprev, weight, wcomp, aw, loop_w, emb8, ab8)


def _final(parts, lprev):
    return pl.pallas_call(
        _final_body,
        grid=(N // _BD,),
        in_specs=_parts_specs() + [
            pl.BlockSpec((_BD, D), lambda i: (i, 0)),
        ],
        out_specs=pl.BlockSpec((_BD, D), lambda i: (i, 0)),
        out_shape=jax.ShapeDtypeStruct((N, D), jnp.float32),
    )(parts, parts, lprev)


_LANE_DNUMS = lax.GatherDimensionNumbers(
    offset_dims=(), collapsed_slice_dims=(0,), start_index_map=(0,))


def _lane_tree_sum(v):
    lanes = lax.iota(jnp.int32, 16)
    for k in (8, 4, 2, 1):
        perm = (lanes + k) & 15
        v = v + lax.gather(v, perm[:, None], _LANE_DNUMS, (1,),
                           mode=lax.GatherScatterMode.PROMISE_IN_BOUNDS)
    return v


def _gate_body(ps_hbm, pd_hbm, cdf_hbm, bv_hbm, bb_hbm,
               src_hbm, dst_hbm, cdi_hbm, s_hbm,
               psr, pdr, srcv, dstv, cdv, cdloc, bvv, bbv, sbuf,
               sem1, sem2):
    cid = lax.axis_index("c")
    sid = lax.axis_index("s")
    wid = cid * 16 + sid
    pltpu.sync_copy(bv_hbm, bvv)
    pltpu.sync_copy(bb_hbm, bbv)
    pltpu.sync_copy(cdf_hbm, cdloc)
    lanes = lax.iota(jnp.int32, 16)
    zero16 = jnp.zeros((16,), jnp.float32)
    lane0 = lanes == 0

    def chunk_body(c, carry):
        base = wid * (GATE_CHUNKS * CHUNK) + c * CHUNK
        pltpu.sync_copy(src_hbm.at[pl.ds(base, CHUNK)], srcv)
        pltpu.sync_copy(dst_hbm.at[pl.ds(base, CHUNK)], dstv)
        pltpu.sync_copy(cdi_hbm.at[pl.ds(base, CHUNK)], cdv)
        cp1 = pltpu.async_copy(ps_hbm.at[srcv], psr, sem1)
        cp2 = pltpu.async_copy(pd_hbm.at[dstv], pdr, sem2)
        cp1.wait()
        cp2.wait()

        @plsc.parallel_loop(0, CHUNK, unroll=4)
        def edge_body(e):
            esplat = jnp.broadcast_to(e, (16,)).astype(jnp.int32)
            crow = plsc.load_gather(cdv, [esplat])
            cbase = crow * DG + lanes
            acc = zero16
            for j in range(13):
                sl = pl.ds(16 * j, 16)
                cdvals = plsc.load_gather(cdloc, [cbase + 16 * j])
                z = psr[e, sl] + pdr[e, sl] + cdvals
                acc = acc + jnp.maximum(z, 0.0) * bvv[sl]
            s = _lane_tree_sum(acc) + bbv[...]
            plsc.store_scatter(sbuf, [esplat], s, mask=lane0)
        pltpu.sync_copy(sbuf, s_hbm.at[pl.ds(base, CHUNK)])
        return carry

    lax.fori_loop(0, GATE_CHUNKS, chunk_body, 0)


def _scatter_body(xra_hbm, xrb_hbm, s_hbm, dst_hbm, msgi_hbm, zeros_hbm,
                  out_hbm, msgr, dstv, msgv, sv, agg, sem1):
    cid = lax.axis_index("c")
    sid = lax.axis_index("s")

    @pl.when(sid != 15)
    def _():
        pltpu.sync_copy(zeros_hbm.at[pl.ds(sid * STRIPE, STRIPE)],
                        agg.at[pl.ds(sid * STRIPE, STRIPE)])

    @pl.when(sid == 15)
    def _():
        pltpu.sync_copy(zeros_hbm.at[pl.ds(15 * STRIPE, STRIPE_LAST)],
                        agg.at[pl.ds(15 * STRIPE, STRIPE_LAST)])

    plsc.subcore_barrier()

    def chunk_body(c, carry):
        base = sid * (SCAT_CHUNKS * CHUNK) + c * CHUNK
        pltpu.sync_copy(dst_hbm.at[pl.ds(base, CHUNK)], dstv)
        pltpu.sync_copy(msgi_hbm.at[pl.ds(base, CHUNK)], msgv)
        pltpu.sync_copy(s_hbm.at[pl.ds(base, CHUNK)], sv)

        @pl.when(cid == 0)
        def _():
            pltpu.async_copy(xra_hbm.at[msgv], msgr, sem1).wait()

        @pl.when(cid == 1)
        def _():
            pltpu.async_copy(xrb_hbm.at[msgv], msgr, sem1).wait()

        for g in range(CHUNK // 16):
            sl = pl.ds(16 * g, 16)
            sv[sl] = 1.0 / (1.0 + jnp.exp(-sv[sl]))

        @plsc.parallel_loop(0, CHUNK, unroll=8)
        def edge_body(e):
            esplat = jnp.broadcast_to(e, (16,)).astype(jnp.int32)
            a = plsc.load_gather(sv, [esplat])
            for j in range(DH // 16):
                sl = pl.ds(16 * j, 16)
                msgr[e, sl] = msgr[e, sl] * a
        pltpu.sync_copy(msgr, agg.at[dstv], add=True)
        return carry

    lax.fori_loop(0, SCAT_CHUNKS, chunk_body, 0)
    plsc.subcore_barrier()

    @pl.when(sid != 15)
    def _():
        pltpu.sync_copy(
            agg.at[pl.ds(sid * STRIPE, STRIPE)],
            out_hbm.at[pl.ds(cid * AGG_ROWS + sid * STRIPE, STRIPE)])

    @pl.when(sid == 15)
    def _():
        pltpu.sync_copy(
            agg.at[pl.ds(15 * STRIPE, STRIPE_LAST)],
            out_hbm.at[pl.ds(cid * AGG_ROWS + 15 * STRIPE, STRIPE_LAST)])


_SC_KERNELS = {}


def _get_sc_kernels():
    if _SC_KERNELS:
        return _SC_KERNELS["gate"], _SC_KERNELS["scatter"]
    mesh = plsc.VectorSubcoreMesh(core_axis_name="c", subcore_axis_name="s")
    gate = pl.kernel(
        _gate_body,
        mesh=mesh,
        compiler_params=pltpu.CompilerParams(needs_layout_passes=False),
        out_type=jax.ShapeDtypeStruct((EP,), jnp.float32),
        scratch_types=[
            pltpu.VMEM((CHUNK, DG), jnp.float32),
            pltpu.VMEM((CHUNK, DG), jnp.float32),
            pltpu.VMEM((CHUNK,), jnp.int32),
            pltpu.VMEM((CHUNK,), jnp.int32),
            pltpu.VMEM((CHUNK,), jnp.int32),
            pltpu.VMEM((16 * DG,), jnp.float32),
            pltpu.VMEM((DG,), jnp.float32),
            pltpu.VMEM((16,), jnp.float32),
            pltpu.VMEM((CHUNK,), jnp.float32),
            pltpu.SemaphoreType.DMA,
            pltpu.SemaphoreType.DMA,
        ],
    )
    mesh2 = plsc.VectorSubcoreMesh(core_axis_name="c", subcore_axis_name="s")
    scatter = pl.kernel(
        _scatter_body,
        mesh=mesh2,
        compiler_params=pltpu.CompilerParams(needs_layout_passes=False),
        out_type=jax.ShapeDtypeStruct((2 * AGG_ROWS, DH), jnp.float32),
        scratch_types=[
            pltpu.VMEM((CHUNK, DH), jnp.float32),
            pltpu.VMEM((CHUNK,), jnp.int32),
            pltpu.VMEM((CHUNK,), jnp.int32),
            pltpu.VMEM((CHUNK,), jnp.float32),
            pltpu.VMEM_SHARED((AGG_ROWS, DH), jnp.float32),
            pltpu.SemaphoreType.DMA,
        ],
    )
    _SC_KERNELS["gate"] = gate
    _SC_KERNELS["scatter"] = scatter
    return gate, scatter


def kernel(x, edge_index, edge_type, edge_label, attn_rel_emb,
           weight0, w_comp0, loop_w0, A_w0, A_b0, B_w0, B_b0,
           weight1, w_comp1, loop_w1, A_w1, A_b1, B_w1, B_b1):
    src = edge_index[0]
    dst = edge_index[1]
    pad = EP - E
    srcp = jnp.concatenate([src, jnp.zeros((pad,), jnp.int32)])
    dstp = jnp.concatenate([dst, jnp.full((pad,), DUMP_ROW, jnp.int32)])
    etp = jnp.concatenate([edge_type, jnp.zeros((pad,), jnp.int32)])
    elp = jnp.concatenate([edge_label, jnp.zeros((pad,), jnp.int32)])
    msgi = etp * N + srcp
    cdi = etp * NREL + elp
    zeros_agg = jnp.zeros((AGG_ROWS, DH), jnp.float32)
    emb8 = jnp.zeros((8, 32), jnp.float32).at[0:NREL, :].set(attn_rel_emb)

    gate, scatter = _get_sc_kernels()

    def layer_edges(xra, xrb, ps, pd, cd, bw, bb):
        bv = jnp.zeros((DG,), jnp.float32).at[0:D].set(bw[:, 0])
        bbs = jnp.broadcast_to(bb, (16,)).astype(jnp.float32)
        cdf = cd.reshape(16 * DG)
        s = gate(ps, pd, cdf, bv, bbs, srcp, dstp, cdi)
        return scatter(xra.reshape(NREL * N, DH), xrb.reshape(NREL * N, DH),
                       s, dstp, msgi, zeros_agg)

    ab80 = jnp.zeros((8, D), jnp.float32).at[0, :].set(A_b0)
    ab81 = jnp.zeros((8, D), jnp.float32).at[0, :].set(A_b1)

    xra1, xrb1, ps1, pd1, lp1, cd1 = _dense1(
        x, weight0, w_comp0, A_w0, loop_w0, emb8, ab80)
    parts1 = layer_edges(xra1, xrb1, ps1, pd1, cd1, B_w0, B_b0)
    xra2, xrb2, ps2, pd2, lp2, cd2 = _dense2(
        parts1, lp1, weight1, w_comp1, A_w1, loop_w1, emb8, ab81)
    parts2 = layer_edges(xra2, xrb2, ps2, pd2, cd2, B_w1, B_b1)
    return _final(parts2, lp2)


# speedup vs baseline: 1.1752x; 1.1752x over previous
import jax
import jax.numpy as jnp
from jax import lax
from jax.experimental import pallas as pl
from jax.experimental.pallas import tpu as pltpu
from jax.experimental.pallas import tpu_sc as plsc

N = 10000
E = 160000
D = 200
DG = 256
DH = 128
NREL = 3
CHUNK = 128
GATE_CHUNKS = 40
EP = 32 * GATE_CHUNKS * CHUNK
SCAT_CHUNKS = EP // (16 * CHUNK)
AGG_ROWS = 11000
STRIPE = 688
STRIPE_LAST = AGG_ROWS - 15 * STRIPE
DUMP_ROW = 10008

_BD = 1000


def _pad_cols(v, width):
    return jnp.concatenate(
        [v, jnp.zeros((v.shape[0], width - v.shape[1]), jnp.float32)], axis=1)


def _dense_products(h, w_ref, wc_ref, aw_ref, lw_ref, emb_ref, ab_ref,
                    xra_ref, xrb_ref, ps_ref, pd_ref, lp_ref, cd_ref):
    hw = [jnp.dot(h, w_ref[b], preferred_element_type=jnp.float32)
          for b in range(4)]
    for r in range(NREL):
        xr_r = (wc_ref[r, 0] * hw[0] + wc_ref[r, 1] * hw[1]
                + wc_ref[r, 2] * hw[2] + wc_ref[r, 3] * hw[3])
        xra_ref[r, :, :] = xr_r[:, 0:DH]
        xrb_ref[r, :, :] = _pad_cols(xr_r[:, DH:D], DH)
    ps_ref[...] = _pad_cols(jnp.dot(h, aw_ref[0:D, :],
                                    preferred_element_type=jnp.float32), DG)
    pd_ref[...] = _pad_cols(jnp.dot(h, aw_ref[D:2 * D, :],
                                    preferred_element_type=jnp.float32), DG)
    lp_ref[...] = jnp.dot(h, lw_ref[...], preferred_element_type=jnp.float32)
    emb = emb_ref[0:NREL, :]
    ca = jnp.dot(emb, aw_ref[2 * D:2 * D + 32, :],
                 preferred_element_type=jnp.float32)
    cb = jnp.dot(emb, aw_ref[2 * D + 32:2 * D + 64, :],
                 preferred_element_type=jnp.float32)
    cd = ca[:, None, :] + cb[None, :, :] + ab_ref[0, :][None, None, :]
    cd16 = jnp.concatenate(
        [cd.reshape(9, D), jnp.zeros((16 - 9, D), jnp.float32)], axis=0)
    cd_ref[...] = _pad_cols(cd16, DG)


def _dense1_body(h_ref, w_ref, wc_ref, aw_ref, lw_ref, emb_ref, ab_ref,
                 xra_ref, xrb_ref, ps_ref, pd_ref, lp_ref, cd_ref):
    _dense_products(h_ref[...], w_ref, wc_ref, aw_ref, lw_ref, emb_ref,
                    ab_ref, xra_ref, xrb_ref, ps_ref, pd_ref, lp_ref, cd_ref)


def _prev_h(pa_ref, pb_ref, lprev_ref):
    return jnp.maximum(
        jnp.concatenate([pa_ref[...], pb_ref[:, 0:D - DH]], axis=1)
        + lprev_ref[...], 0.0)


def _dense2_body(pa_ref, pb_ref, lprev_ref, w_ref, wc_ref, aw_ref, lw_ref,
                 emb_ref, ab_ref, xra_ref, xrb_ref, ps_ref, pd_ref, lp_ref,
                 cd_ref):
    _dense_products(_prev_h(pa_ref, pb_ref, lprev_ref), w_ref, wc_ref, aw_ref,
                    lw_ref, emb_ref, ab_ref, xra_ref, xrb_ref, ps_ref, pd_ref,
                    lp_ref, cd_ref)


def _final_body(pa_ref, pb_ref, lprev_ref, out_ref):
    out_ref[...] = _prev_h(pa_ref, pb_ref, lprev_ref)


def _dense_out_shapes():
    return [
        jax.ShapeDtypeStruct((NREL, N, DH), jnp.float32),
        jax.ShapeDtypeStruct((NREL, N, DH), jnp.float32),
        jax.ShapeDtypeStruct((N, DG), jnp.float32),
        jax.ShapeDtypeStruct((N, DG), jnp.float32),
        jax.ShapeDtypeStruct((N, D), jnp.float32),
        jax.ShapeDtypeStruct((16, DG), jnp.float32),
    ]


def _dense_out_specs():
    return [
        pl.BlockSpec((NREL, _BD, DH), lambda i: (0, i, 0)),
        pl.BlockSpec((NREL, _BD, DH), lambda i: (0, i, 0)),
        pl.BlockSpec((_BD, DG), lambda i: (i, 0)),
        pl.BlockSpec((_BD, DG), lambda i: (i, 0)),
        pl.BlockSpec((_BD, D), lambda i: (i, 0)),
        pl.BlockSpec((16, DG), lambda i: (0, 0)),
    ]


def _weight_specs():
    return [
        pl.BlockSpec((4, D, D), lambda i: (0, 0, 0)),
        pl.BlockSpec(memory_space=pltpu.SMEM),
        pl.BlockSpec((2 * D + 64, D), lambda i: (0, 0)),
        pl.BlockSpec((D, D), lambda i: (0, 0)),
        pl.BlockSpec((8, 32), lambda i: (0, 0)),
        pl.BlockSpec((8, D), lambda i: (0, 0)),
    ]


def _parts_specs():
    return [
        pl.BlockSpec((_BD, DH), lambda i: (i, 0)),
        pl.BlockSpec((_BD, DH), lambda i: (AGG_ROWS // _BD + i, 0)),
    ]


def _dense1(h, weight, wcomp, aw, loop_w, emb8, ab8):
    return pl.pallas_call(
        _dense1_body,
        grid=(N // _BD,),
        in_specs=[pl.BlockSpec((_BD, D), lambda i: (i, 0))] + _weight_specs(),
        out_specs=_dense_out_specs(),
        out_shape=_dense_out_shapes(),
    )(h, weight, wcomp, aw, loop_w, emb8, ab8)


def _dense2(parts, lprev, weight, wcomp, aw, loop_w, emb8, ab8):
    return pl.pallas_call(
        _dense2_body,
        grid=(N // _BD,),
        in_specs=_parts_specs() + [
            pl.BlockSpec((_BD, D), lambda i: (i, 0)),
        ] + _weight_specs(),
        out_specs=_dense_out_specs(),
        out_shape=_dense_out_shapes(),
    )(parts, parts, lprev, weight, wcomp, aw, loop_w, emb8, ab8)


def _final(parts, lprev):
    return pl.pallas_call(
        _final_body,
        grid=(N // _BD,),
        in_specs=_parts_specs() + [
            pl.BlockSpec((_BD, D), lambda i: (i, 0)),
        ],
        out_specs=pl.BlockSpec((_BD, D), lambda i: (i, 0)),
        out_shape=jax.ShapeDtypeStruct((N, D), jnp.float32),
    )(parts, parts, lprev)


_LANE_DNUMS = lax.GatherDimensionNumbers(
    offset_dims=(), collapsed_slice_dims=(0,), start_index_map=(0,))


def _lane_tree_sum(v):
    lanes = lax.iota(jnp.int32, 16)
    for k in (8, 4, 2, 1):
        perm = (lanes + k) & 15
        v = v + lax.gather(v, perm[:, None], _LANE_DNUMS, (1,),
                           mode=lax.GatherScatterMode.PROMISE_IN_BOUNDS)
    return v


def _gate_body(ps_hbm, pd_hbm, cdf_hbm, bv_hbm, bb_hbm,
               src_hbm, dst_hbm, cdi_hbm, s_hbm,
               psr, pdr, srcv, dstv, cdv, cdloc, bvv, bbv, sbuf,
               sem1, sem2):
    cid = lax.axis_index("c")
    sid = lax.axis_index("s")
    wid = cid * 16 + sid
    pltpu.sync_copy(bv_hbm, bvv)
    pltpu.sync_copy(bb_hbm, bbv)
    pltpu.sync_copy(cdf_hbm, cdloc)
    lanes = lax.iota(jnp.int32, 16)
    zero16 = jnp.zeros((16,), jnp.float32)
    lane0 = lanes == 0

    def chunk_body(c, carry):
        base = wid * (GATE_CHUNKS * CHUNK) + c * CHUNK
        pltpu.sync_copy(src_hbm.at[pl.ds(base, CHUNK)], srcv)
        pltpu.sync_copy(dst_hbm.at[pl.ds(base, CHUNK)], dstv)
        pltpu.sync_copy(cdi_hbm.at[pl.ds(base, CHUNK)], cdv)
        cp1 = pltpu.async_copy(ps_hbm.at[srcv], psr, sem1)
        cp2 = pltpu.async_copy(pd_hbm.at[dstv], pdr, sem2)
        cp1.wait()
        cp2.wait()

        @plsc.parallel_loop(0, CHUNK, unroll=2)
        def edge_body(e):
            esplat = jnp.broadcast_to(e, (16,)).astype(jnp.int32)
            crow = plsc.load_gather(cdv, [esplat])
            cbase = crow * DG + lanes
            acc = zero16
            for j in range(13):
                sl = pl.ds(16 * j, 16)
                cdvals = plsc.load_gather(cdloc, [cbase + 16 * j])
                z = psr[e, sl] + pdr[e, sl] + cdvals
                acc = acc + jnp.maximum(z, 0.0) * bvv[sl]
            s = _lane_tree_sum(acc) + bbv[...]
            plsc.store_scatter(sbuf, [esplat], s, mask=lane0)
        pltpu.sync_copy(sbuf, s_hbm.at[pl.ds(base, CHUNK)])
        return carry

    lax.fori_loop(0, GATE_CHUNKS, chunk_body, 0)


def _scatter_body(xra_hbm, xrb_hbm, s_hbm, dst_hbm, msgi_hbm, zeros_hbm,
                  out_hbm, msgr, dstv, msgv, sv, msgr2, dstv2, msgv2, sv2,
                  agg, sem1, sem2):
    cid = lax.axis_index("c")
    sid = lax.axis_index("s")

    @pl.when(sid != 15)
    def _():
        pltpu.sync_copy(zeros_hbm.at[pl.ds(sid * STRIPE, STRIPE)],
                        agg.at[pl.ds(sid * STRIPE, STRIPE)])

    @pl.when(sid == 15)
    def _():
        pltpu.sync_copy(zeros_hbm.at[pl.ds(15 * STRIPE, STRIPE_LAST)],
                        agg.at[pl.ds(15 * STRIPE, STRIPE_LAST)])

    plsc.subcore_barrier()

    def fire(c, msgv_, dstv_, sv_, msgr_, sem_):
        base = sid * (SCAT_CHUNKS * CHUNK) + c * CHUNK
        pltpu.sync_copy(dst_hbm.at[pl.ds(base, CHUNK)], dstv_)
        pltpu.sync_copy(msgi_hbm.at[pl.ds(base, CHUNK)], msgv_)
        pltpu.sync_copy(s_hbm.at[pl.ds(base, CHUNK)], sv_)
        cpa = pltpu.make_async_copy(xra_hbm.at[msgv_], msgr_, sem_)
        cpb = pltpu.make_async_copy(xrb_hbm.at[msgv_], msgr_, sem_)

        @pl.when(cid == 0)
        def _():
            cpa.start()

        @pl.when(cid == 1)
        def _():
            cpb.start()

    def consume(msgv_, dstv_, sv_, msgr_, sem_):
        pltpu.make_async_copy(xra_hbm.at[msgv_], msgr_, sem_).wait()
        for g in range(CHUNK // 16):
            sl = pl.ds(16 * g, 16)
            sv_[sl] = 1.0 / (1.0 + jnp.exp(-sv_[sl]))

        @plsc.parallel_loop(0, CHUNK, unroll=4)
        def edge_body(e):
            esplat = jnp.broadcast_to(e, (16,)).astype(jnp.int32)
            a = plsc.load_gather(sv_, [esplat])
            for j in range(DH // 16):
                sl = pl.ds(16 * j, 16)
                msgr_[e, sl] = msgr_[e, sl] * a
        pltpu.sync_copy(msgr_, agg.at[dstv_], add=True)

    npairs = SCAT_CHUNKS // 2
    fire(0, msgv, dstv, sv, msgr, sem1)

    def pair_body(p, carry):
        c0 = 2 * p
        fire(c0 + 1, msgv2, dstv2, sv2, msgr2, sem2)
        consume(msgv, dstv, sv, msgr, sem1)

        @pl.when(p + 1 < npairs)
        def _():
            fire(c0 + 2, msgv, dstv, sv, msgr, sem1)

        consume(msgv2, dstv2, sv2, msgr2, sem2)
        return carry

    lax.fori_loop(0, npairs, pair_body, 0)
    plsc.subcore_barrier()

    @pl.when(sid != 15)
    def _():
        pltpu.sync_copy(
            agg.at[pl.ds(sid * STRIPE, STRIPE)],
            out_hbm.at[pl.ds(cid * AGG_ROWS + sid * STRIPE, STRIPE)])

    @pl.when(sid == 15)
    def _():
        pltpu.sync_copy(
            agg.at[pl.ds(15 * STRIPE, STRIPE_LAST)],
            out_hbm.at[pl.ds(cid * AGG_ROWS + 15 * STRIPE, STRIPE_LAST)])


_SC_KERNELS = {}


def _get_sc_kernels():
    if _SC_KERNELS:
        return _SC_KERNELS["gate"], _SC_KERNELS["scatter"]
    mesh = plsc.VectorSubcoreMesh(core_axis_name="c", subcore_axis_name="s")
    gate = pl.kernel(
        _gate_body,
        mesh=mesh,
        compiler_params=pltpu.CompilerParams(needs_layout_passes=False),
        out_type=jax.ShapeDtypeStruct((EP,), jnp.float32),
        scratch_types=[
            pltpu.VMEM((CHUNK, DG), jnp.float32),
            pltpu.VMEM((CHUNK, DG), jnp.float32),
            pltpu.VMEM((CHUNK,), jnp.int32),
            pltpu.VMEM((CHUNK,), jnp.int32),
            pltpu.VMEM((CHUNK,), jnp.int32),
            pltpu.VMEM((16 * DG,), jnp.float32),
            pltpu.VMEM((DG,), jnp.float32),
            pltpu.VMEM((16,), jnp.float32),
            pltpu.VMEM((CHUNK,), jnp.float32),
            pltpu.SemaphoreType.DMA,
            pltpu.SemaphoreType.DMA,
        ],
    )
    mesh2 = plsc.VectorSubcoreMesh(core_axis_name="c", subcore_axis_name="s")
    scatter = pl.kernel(
        _scatter_body,
        mesh=mesh2,
        compiler_params=pltpu.CompilerParams(needs_layout_passes=False),
        out_type=jax.ShapeDtypeStruct((2 * AGG_ROWS, DH), jnp.float32),
        scratch_types=[
            pltpu.VMEM((CHUNK, DH), jnp.float32),
            pltpu.VMEM((CHUNK,), jnp.int32),
            pltpu.VMEM((CHUNK,), jnp.int32),
            pltpu.VMEM((CHUNK,), jnp.float32),
            pltpu.VMEM((CHUNK, DH), jnp.float32),
            pltpu.VMEM((CHUNK,), jnp.int32),
            pltpu.VMEM((CHUNK,), jnp.int32),
            pltpu.VMEM((CHUNK,), jnp.float32),
            pltpu.VMEM_SHARED((AGG_ROWS, DH), jnp.float32),
            pltpu.SemaphoreType.DMA,
            pltpu.SemaphoreType.DMA,
        ],
    )
    _SC_KERNELS["gate"] = gate
    _SC_KERNELS["scatter"] = scatter
    return gate, scatter


def kernel(x, edge_index, edge_type, edge_label, attn_rel_emb,
           weight0, w_comp0, loop_w0, A_w0, A_b0, B_w0, B_b0,
           weight1, w_comp1, loop_w1, A_w1, A_b1, B_w1, B_b1):
    src = edge_index[0]
    dst = edge_index[1]
    pad = EP - E
    srcp = jnp.concatenate([src, jnp.zeros((pad,), jnp.int32)])
    dstp = jnp.concatenate([dst, jnp.full((pad,), DUMP_ROW, jnp.int32)])
    etp = jnp.concatenate([edge_type, jnp.zeros((pad,), jnp.int32)])
    elp = jnp.concatenate([edge_label, jnp.zeros((pad,), jnp.int32)])
    msgi = etp * N + srcp
    cdi = etp * NREL + elp
    zeros_agg = jnp.zeros((AGG_ROWS, DH), jnp.float32)
    emb8 = jnp.zeros((8, 32), jnp.float32).at[0:NREL, :].set(attn_rel_emb)

    gate, scatter = _get_sc_kernels()

    def layer_edges(xra, xrb, ps, pd, cd, bw, bb):
        bv = jnp.zeros((DG,), jnp.float32).at[0:D].set(bw[:, 0])
        bbs = jnp.broadcast_to(bb, (16,)).astype(jnp.float32)
        cdf = cd.reshape(16 * DG)
        s = gate(ps, pd, cdf, bv, bbs, srcp, dstp, cdi)
        return scatter(xra.reshape(NREL * N, DH), xrb.reshape(NREL * N, DH),
                       s, dstp, msgi, zeros_agg)

    ab80 = jnp.zeros((8, D), jnp.float32).at[0, :].set(A_b0)
    ab81 = jnp.zeros((8, D), jnp.float32).at[0, :].set(A_b1)

    xra1, xrb1, ps1, pd1, lp1, cd1 = _dense1(
        x, weight0, w_comp0, A_w0, loop_w0, emb8, ab80)
    parts1 = layer_edges(xra1, xrb1, ps1, pd1, cd1, B_w0, B_b0)
    xra2, xrb2, ps2, pd2, lp2, cd2 = _dense2(
        parts1, lp1, weight1, w_comp1, A_w1, loop_w1, emb8, ab81)
    parts2 = layer_edges(xra2, xrb2, ps2, pd2, cd2, B_w1, B_b1)
    return _final(parts2, lp2)
